# baseline (speedup 1.0000x reference)
import functools

import jax
import jax.numpy as jnp
from jax import lax
from jax.experimental import pallas as pl
from jax.experimental.pallas import tpu as pltpu
from jax.experimental.pallas import tpu_sc as plsc

L = 16


def _sc_add_kernel(B, S, D, R, n_workers):
    s_per_w = S // n_workers
    n_blocks = s_per_w // R
    vecs = (R * D) // L

    mesh = plsc.VectorSubcoreMesh(core_axis_name="c", subcore_axis_name="s")

    @functools.partial(
        pl.kernel,
        mesh=mesh,
        out_type=jax.ShapeDtypeStruct((B, S, D), jnp.float32),
        scratch_types=[
            pltpu.VMEM((R, D), jnp.float32),
            pltpu.VMEM((R, D), jnp.float32),
        ],
    )
    def k(x_hbm, p_hbm, o_hbm, pbuf, xbuf):
        nc = 2
        wid = lax.axis_index("s") * nc + lax.axis_index("c")
        base0 = wid * s_per_w

        def blk_body(blk, _):
            base = base0 + blk * R
            pltpu.sync_copy(p_hbm.at[pl.ds(base, R)], pbuf)
            for b in range(B):
                pltpu.sync_copy(x_hbm.at[b, pl.ds(base, R)], xbuf)

                def add_body(i, _):
                    r = i // (D // L)
                    c = (i % (D // L)) * L
                    xbuf[r, pl.ds(c, L)] = xbuf[r, pl.ds(c, L)] + pbuf[r, pl.ds(c, L)]
                    return 0

                lax.fori_loop(0, vecs, add_body, 0)
                pltpu.sync_copy(xbuf, o_hbm.at[b, pl.ds(base, R)])
            return 0

        lax.fori_loop(0, n_blocks, blk_body, 0)

    return k


def kernel(x, pos_table):
    B, S, D = x.shape
    k = _sc_add_kernel(B, S, D, R=32, n_workers=32)
    return k(x, pos_table[:S])


# speedup vs baseline: 3.3660x; 3.3660x over previous
import functools

import jax
import jax.numpy as jnp
from jax import lax
from jax.experimental import pallas as pl
from jax.experimental.pallas import tpu as pltpu
from jax.experimental.pallas import tpu_sc as plsc

L = 16


def _sc_add_kernel(B, S, D, R, n_workers):
    s_per_w = S // n_workers
    n_blocks = s_per_w // R
    assert n_blocks % 2 == 0 and S % n_workers == 0 and s_per_w % R == 0
    assert D % L == 0

    mesh = plsc.VectorSubcoreMesh(core_axis_name="c", subcore_axis_name="s")

    @functools.partial(
        pl.kernel,
        mesh=mesh,
        out_type=jax.ShapeDtypeStruct((B, S, D), jnp.float32),
        scratch_types=[
            pltpu.VMEM((R, D), jnp.float32),
            pltpu.VMEM((R, D), jnp.float32),
            pltpu.VMEM((B, R, D), jnp.float32),
            pltpu.VMEM((B, R, D), jnp.float32),
            pltpu.VMEM((B, R, D), jnp.float32),
            pltpu.VMEM((B, R, D), jnp.float32),
            pltpu.SemaphoreType.DMA,
            pltpu.SemaphoreType.DMA,
            pltpu.SemaphoreType.DMA((B,)),
            pltpu.SemaphoreType.DMA((B,)),
            pltpu.SemaphoreType.DMA,
            pltpu.SemaphoreType.DMA,
        ],
    )
    def k(x_hbm, p_hbm, o_hbm, pbuf_a, pbuf_b, xin_a, xin_b, xout_a, xout_b,
          semp_a, semp_b, semin_a, semin_b, semout_a, semout_b):
        nc = 2
        wid = lax.axis_index("s") * nc + lax.axis_index("c")
        base0 = wid * s_per_w

        def fire_in(blk, pbuf, xin, semp, semin):
            base = base0 + blk * R
            pltpu.async_copy(p_hbm.at[pl.ds(base, R)], pbuf, semp)
            for b in range(B):
                pltpu.async_copy(x_hbm.at[b, pl.ds(base, R)], xin.at[b], semin.at[b])

        def process(blk, pbuf, xin, xout, semp, semin, semout):
            base = base0 + blk * R
            pltpu.make_async_copy(p_hbm.at[pl.ds(base, R)], pbuf, semp).wait()
            for b in range(B):
                pltpu.make_async_copy(
                    x_hbm.at[b, pl.ds(base, R)], xin.at[b], semin.at[b]).wait()

            @pl.when(blk >= 2)
            def _():
                for b in range(B):
                    pltpu.make_async_copy(
                        xout.at[b], o_hbm.at[b, pl.ds(base, R)], semout).wait()

            for r in range(R):
                @plsc.parallel_loop(0, D // L, unroll=4)
                def _(i):
                    c = i * L
                    pv = pbuf[r, pl.ds(c, L)]
                    for b in range(B):
                        xout[b, r, pl.ds(c, L)] = xin[b, r, pl.ds(c, L)] + pv

            for b in range(B):
                pltpu.async_copy(xout.at[b], o_hbm.at[b, pl.ds(base, R)], semout)

            @pl.when(blk + 2 < n_blocks)
            def _():
                fire_in(blk + 2, pbuf, xin, semp, semin)

        fire_in(0, pbuf_a, xin_a, semp_a, semin_a)
        fire_in(1, pbuf_b, xin_b, semp_b, semin_b)

        def pair_body(j, _):
            process(2 * j, pbuf_a, xin_a, xout_a, semp_a, semin_a, semout_a)
            process(2 * j + 1, pbuf_b, xin_b, xout_b, semp_b, semin_b, semout_b)
            return 0

        lax.fori_loop(0, n_blocks // 2, pair_body, 0)

        for blk, xout, semout in ((n_blocks - 2, xout_a, semout_a),
                                  (n_blocks - 1, xout_b, semout_b)):
            base = base0 + blk * R
            for b in range(B):
                pltpu.make_async_copy(
                    xout.at[b], o_hbm.at[b, pl.ds(base, R)], semout).wait()

    return k


def kernel(x, pos_table):
    B, S, D = x.shape
    k = _sc_add_kernel(B, S, D, R=8, n_workers=32)
    return k(x, pos_table[:S])


# speedup vs baseline: 3.4481x; 1.0244x over previous
import functools

import jax
import jax.numpy as jnp
from jax import lax
from jax.experimental import pallas as pl
from jax.experimental.pallas import tpu as pltpu
from jax.experimental.pallas import tpu_sc as plsc

L = 16


def _sc_add_kernel(B, S, D, R, n_workers):
    s_per_w = S // n_workers
    n_blocks = s_per_w // R
    assert n_blocks % 2 == 0 and S % n_workers == 0 and s_per_w % R == 0
    assert D % L == 0

    mesh = plsc.VectorSubcoreMesh(core_axis_name="c", subcore_axis_name="s")

    @functools.partial(
        pl.kernel,
        mesh=mesh,
        out_type=jax.ShapeDtypeStruct((B, S, D), jnp.float32),
        scratch_types=[
            pltpu.VMEM((R, D), jnp.float32),
            pltpu.VMEM((R, D), jnp.float32),
            pltpu.VMEM((B, R, D), jnp.float32),
            pltpu.VMEM((B, R, D), jnp.float32),
            pltpu.VMEM((B, R, D), jnp.float32),
            pltpu.VMEM((B, R, D), jnp.float32),
            pltpu.SemaphoreType.DMA,
            pltpu.SemaphoreType.DMA,
            pltpu.SemaphoreType.DMA((B,)),
            pltpu.SemaphoreType.DMA((B,)),
            pltpu.SemaphoreType.DMA,
            pltpu.SemaphoreType.DMA,
        ],
    )
    def k(x_hbm, p_hbm, o_hbm, pbuf_a, pbuf_b, xin_a, xin_b, xout_a, xout_b,
          semp_a, semp_b, semin_a, semin_b, semout_a, semout_b):
        nc = 2
        wid = lax.axis_index("s") * nc + lax.axis_index("c")
        base0 = wid * s_per_w

        def fire_in(blk, pbuf, xin, semp, semin):
            base = base0 + blk * R
            pltpu.async_copy(p_hbm.at[pl.ds(base, R)], pbuf, semp)
            for b in range(B):
                pltpu.async_copy(x_hbm.at[b, pl.ds(base, R)], xin.at[b], semin.at[b])

        def process(blk, pbuf, xin, xout, semp, semin, semout):
            base = base0 + blk * R
            pltpu.make_async_copy(p_hbm.at[pl.ds(base, R)], pbuf, semp).wait()
            for b in range(B):
                pltpu.make_async_copy(
                    x_hbm.at[b, pl.ds(base, R)], xin.at[b], semin.at[b]).wait()

            @pl.when(blk >= 2)
            def _():
                for b in range(B):
                    pltpu.make_async_copy(
                        xout.at[b], o_hbm.at[b, pl.ds(base, R)], semout).wait()

            @plsc.parallel_loop(0, D // L, unroll=2)
            def _(i):
                c = i * L
                for r in range(R):
                    pv = pbuf[r, pl.ds(c, L)]
                    for b in range(B):
                        xout[b, r, pl.ds(c, L)] = xin[b, r, pl.ds(c, L)] + pv

            for b in range(B):
                pltpu.async_copy(xout.at[b], o_hbm.at[b, pl.ds(base, R)], semout)

            @pl.when(blk + 2 < n_blocks)
            def _():
                fire_in(blk + 2, pbuf, xin, semp, semin)

        fire_in(0, pbuf_a, xin_a, semp_a, semin_a)
        fire_in(1, pbuf_b, xin_b, semp_b, semin_b)

        def pair_body(j, _):
            process(2 * j, pbuf_a, xin_a, xout_a, semp_a, semin_a, semout_a)
            process(2 * j + 1, pbuf_b, xin_b, xout_b, semp_b, semin_b, semout_b)
            return 0

        lax.fori_loop(0, n_blocks // 2, pair_body, 0)

        for blk, xout, semout in ((n_blocks - 2, xout_a, semout_a),
                                  (n_blocks - 1, xout_b, semout_b)):
            base = base0 + blk * R
            for b in range(B):
                pltpu.make_async_copy(
                    xout.at[b], o_hbm.at[b, pl.ds(base, R)], semout).wait()

    return k


def kernel(x, pos_table):
    B, S, D = x.shape
    k = _sc_add_kernel(B, S, D, R=8, n_workers=32)
    return k(x, pos_table[:S])
